# Initial kernel scaffold; baseline (speedup 1.0000x reference)
import jax
import jax.numpy as jnp
from jax.experimental import pallas as pl


def kernel(x, edge_index, W1, b1, adj1, W2, b2, adj2, fc_W, fc_b):
    raise NotImplementedError("write your pallas kernel here")


# speedup vs baseline: 5.0299x; 5.0299x over previous
import functools

import jax
import jax.numpy as jnp
from jax import lax
from jax.experimental import pallas as pl
from jax.experimental.pallas import tpu as pltpu
from jax.experimental.pallas import tpu_sc as plsc

NC = 2
NS = 16
NW = NC * NS


def _sc_gather_scatter_add(table, src, dst, zrows):
    N, D = table.shape
    E = src.shape[0]
    EPW = E // NW
    C = 80
    NCH = EPW // C
    RA = ((N // NS) + 7) // 8 * 8
    RL = N - RA * (NS - 1)
    assert RL > 0 and RA % 8 == 0 and RL % 8 == 0
    mesh = plsc.VectorSubcoreMesh(core_axis_name="c", subcore_axis_name="s")

    @functools.partial(
        pl.kernel,
        out_type=jax.ShapeDtypeStruct((NC, N, D), jnp.float32),
        mesh=mesh,
        scratch_types=[
            pltpu.VMEM((C,), jnp.int32),
            pltpu.VMEM((C,), jnp.int32),
            pltpu.VMEM((C, D), jnp.float32),
            pltpu.VMEM_SHARED((N, D), jnp.float32),
            pltpu.SemaphoreType.DMA,
        ],
    )
    def k(table_hbm, src_hbm, dst_hbm, z_hbm, out_hbm, sidx, didx, rows, acc, sem):
        c = lax.axis_index("c")
        s = lax.axis_index("s")

        @pl.when(s < NS - 1)
        def _():
            pltpu.sync_copy(z_hbm, acc.at[pl.ds(s * RA, RA)])

        @pl.when(s == NS - 1)
        def _():
            pltpu.sync_copy(z_hbm.at[pl.ds(0, RL)],
                            acc.at[pl.ds((NS - 1) * RA, RL)])

        plsc.subcore_barrier()
        base = (c * NS + s) * EPW

        @pl.loop(0, NCH)
        def _(i):
            off = base + i * C
            pltpu.sync_copy(src_hbm.at[pl.ds(off, C)], sidx)
            pltpu.sync_copy(dst_hbm.at[pl.ds(off, C)], didx)
            pltpu.async_copy(table_hbm.at[sidx], rows, sem).wait()
            pltpu.sync_copy(rows, acc.at[didx], add=True)

        plsc.subcore_barrier()

        @pl.when(s < NS - 1)
        def _():
            pltpu.sync_copy(acc.at[pl.ds(s * RA, RA)],
                            out_hbm.at[c, pl.ds(s * RA, RA)])

        @pl.when(s == NS - 1)
        def _():
            pltpu.sync_copy(acc.at[pl.ds((NS - 1) * RA, RL)],
                            out_hbm.at[c, pl.ds((NS - 1) * RA, RL)])

    return k(table, src, dst, zrows)


def _tc_layer(a0, a1, feats, W, b, lam):
    N, D = feats.shape
    R = 1000
    G = N // R

    def body(a0_ref, a1_ref, f_ref, w_ref, b_ref, lam_ref, o_ref):
        z = a0_ref[...] + a1_ref[...] + lam_ref[0, 0] * f_ref[...]
        y = jnp.dot(z, w_ref[...], preferred_element_type=jnp.float32)
        o_ref[...] = jax.nn.sigmoid(y + b_ref[...])

    return pl.pallas_call(
        body,
        grid=(G,),
        in_specs=[
            pl.BlockSpec((R, D), lambda i: (i, 0)),
            pl.BlockSpec((R, D), lambda i: (i, 0)),
            pl.BlockSpec((R, D), lambda i: (i, 0)),
            pl.BlockSpec((D, D), lambda i: (0, 0)),
            pl.BlockSpec((1, D), lambda i: (0, 0)),
            pl.BlockSpec((1, 1), lambda i: (0, 0)),
        ],
        out_specs=pl.BlockSpec((R, D), lambda i: (i, 0)),
        out_shape=jax.ShapeDtypeStruct((N, D), jnp.float32),
    )(a0, a1, feats, W, b, lam)


def _tc_layer_final(a0, a1, feats, W, b, lam, fcw_row, fcb):
    N, D = feats.shape
    R = 1000
    G = N // R

    def body(a0_ref, a1_ref, f_ref, w_ref, b_ref, lam_ref, fcw_ref, fcb_ref,
             out_ref, gr_ref):
        i = pl.program_id(0)
        z = a0_ref[...] + a1_ref[...] + lam_ref[0, 0] * f_ref[...]
        y = jax.nn.sigmoid(
            jnp.dot(z, w_ref[...], preferred_element_type=jnp.float32)
            + b_ref[...])

        @pl.when(i == 0)
        def _():
            gr_ref[...] = jnp.zeros_like(gr_ref)

        gr_ref[...] += jnp.sum(y, axis=0, keepdims=True)

        @pl.when(i == G - 1)
        def _():
            out_ref[...] = (jnp.sum(gr_ref[...] * fcw_ref[...], axis=1,
                                    keepdims=True) + fcb_ref[...])

    blk = lambda r, c: pl.BlockSpec((r, c), lambda i: (i, 0))
    const = lambda r, c: pl.BlockSpec((r, c), lambda i: (0, 0))
    out, gr = pl.pallas_call(
        body,
        grid=(G,),
        in_specs=[
            blk(R, D), blk(R, D), blk(R, D),
            const(D, D), const(1, D), const(1, 1),
            const(1, D), const(1, 1),
        ],
        out_specs=[const(1, 1), const(1, D)],
        out_shape=[
            jax.ShapeDtypeStruct((1, 1), jnp.float32),
            jax.ShapeDtypeStruct((1, D), jnp.float32),
        ],
    )(a0, a1, feats, W, b, lam, fcw_row, fcb)
    return out, gr


def kernel(x, edge_index, W1, b1, adj1, W2, b2, adj2, fc_W, fc_b):
    N, D = x.shape
    src = edge_index[0].astype(jnp.int32)
    dst = edge_index[1].astype(jnp.int32)
    zrows = jnp.zeros((((N // NS) + 7) // 8 * 8, D), jnp.float32)
    b1r = b1.reshape(1, D)
    b2r = b2.reshape(1, D)
    lam1 = adj1.reshape(1, 1).astype(jnp.float32)
    lam2 = adj2.reshape(1, 1).astype(jnp.float32)
    fcw_row = fc_W.reshape(1, D)
    fcb = fc_b.reshape(1, 1)

    p1 = _sc_gather_scatter_add(x, src, dst, zrows)
    f1 = _tc_layer(p1[0], p1[1], x, W1, b1r, lam1)
    p2 = _sc_gather_scatter_add(f1, src, dst, zrows)
    out, gr = _tc_layer_final(p2[0], p2[1], f1, W2, b2r, lam2, fcw_row, fcb)
    return (out, gr)
